# Initial kernel scaffold; baseline (speedup 1.0000x reference)
import jax
import jax.numpy as jnp
from jax.experimental import pallas as pl


def kernel(x, edge_index, W1, att_src1, att_dst1, b1, W2, att_src2, att_dst2, b2):
    raise NotImplementedError("write your pallas kernel here")


# speedup vs baseline: 40.3883x; 40.3883x over previous
import functools

import numpy as np
import jax
import jax.numpy as jnp
from jax import lax
from jax.experimental import pallas as pl
from jax.experimental.pallas import tpu as pltpu
from jax.experimental.pallas import tpu_sc as plsc

N = 10000
NP = 10240
E = 320000
HEADS = 8
HID = 8
OUT_CH = 64
DW = 80
NC, NS = 2, 16
NW = NC * NS
PERW = E // NW
K = 80
CHUNKS = PERW // K
NPT = NP // NS

f32 = jnp.float32
i32 = jnp.int32

_E8 = np.repeat(np.eye(8, dtype=np.float32), 8, axis=1)
_B64 = np.zeros((8, 64), np.float32)
_B64[0, :] = 1.0
_R8 = np.zeros((8, 8), np.float32)
_R8[0, :] = 1.0


def _leaky_exp(z):
    return jnp.exp(jnp.where(z >= 0, z, 0.2 * z))


def _padrows(a):
    return jnp.concatenate([a, jnp.zeros((NP - N, a.shape[1]), f32)], axis=0)


def _tc1_body(x_ref, w_ref, ams_ref, amd_ref, e8_ref,
              hext_ref, accinit_ref, adst_ref):
    h = jnp.dot(x_ref[...], w_ref[...], preferred_element_type=f32)
    a_s = jnp.dot(h, ams_ref[...], preferred_element_type=f32)
    a_d = jnp.dot(h, amd_ref[...], preferred_element_type=f32)
    w = _leaky_exp(a_s + a_d)
    w64 = jnp.dot(w, e8_ref[...], preferred_element_type=f32)
    z8 = jnp.zeros((N, 8), f32)
    hext_ref[...] = _padrows(jnp.concatenate([h, a_s, z8], axis=1))
    accinit_ref[...] = _padrows(jnp.concatenate([h * w64, w, z8], axis=1))
    adst_ref[...] = _padrows(jnp.concatenate([a_d, z8], axis=1))


def _tc2_body(acc_ref, b1_ref, w2_ref, ams_ref, amd_ref, e8_ref, b64_ref, r8_ref,
              hext_ref, accinit_ref, adst_ref):
    a = acc_ref[0:N, :] + acc_ref[NP:NP + N, :]
    den = jnp.dot(a[:, 64:72], e8_ref[...], preferred_element_type=f32)
    o1 = a[:, 0:64] / (den + 1e-16) + b1_ref[...]
    h1 = jnp.where(o1 > 0, o1, jnp.exp(o1) - 1.0)
    h2 = jnp.dot(h1, w2_ref[...], preferred_element_type=f32)
    a_s = jnp.dot(h2, ams_ref[...], preferred_element_type=f32)
    a_d = jnp.dot(h2, amd_ref[...], preferred_element_type=f32)
    w8 = _leaky_exp(a_s + a_d)
    wrep = jnp.dot(w8, r8_ref[...], preferred_element_type=f32)
    w64 = jnp.dot(w8, b64_ref[...], preferred_element_type=f32)
    z8 = jnp.zeros((N, 8), f32)
    hext_ref[...] = _padrows(jnp.concatenate([h2, a_s, z8], axis=1))
    accinit_ref[...] = _padrows(jnp.concatenate([h2 * w64, wrep, z8], axis=1))
    adst_ref[...] = _padrows(jnp.concatenate([a_d, z8], axis=1))


def _tc3_body(acc_ref, b2_ref, b64_ref, out_ref):
    a = acc_ref[0:N, :] + acc_ref[NP:NP + N, :]
    den = jnp.dot(a[:, 64:72], b64_ref[...], preferred_element_type=f32)
    o = a[:, 0:64] / (den + 1e-16) + b2_ref[...]
    m = jnp.max(o, axis=1, keepdims=True)
    ls = o - m
    out_ref[...] = ls - jnp.log(jnp.sum(jnp.exp(ls), axis=1, keepdims=True))


def _make_sc_edge(H):
    mesh = plsc.VectorSubcoreMesh(core_axis_name="c", subcore_axis_name="s")

    @functools.partial(
        pl.kernel,
        out_type=jax.ShapeDtypeStruct((NC * NP, DW), f32),
        mesh=mesh,
        compiler_params=pltpu.CompilerParams(
            needs_layout_passes=False,
            use_tc_tiling_on_sc=False,
        ),
        scratch_types=[
            pltpu.VMEM_SHARED((NP, DW), f32),
            pltpu.VMEM((K,), i32),
            pltpu.VMEM((K,), i32),
            pltpu.VMEM((K, DW), f32),
            pltpu.VMEM((K, 16), f32),
            pltpu.VMEM((K * 8,), f32),
            pltpu.VMEM((K, DW), f32),
            pltpu.VMEM((NPT, DW), f32),
            pltpu.SemaphoreType.DMA,
            pltpu.SemaphoreType.DMA,
        ],
    )
    def kern(hext, adstt, srci, dsti, accinit, out,
             acc_sh, sidx, didx, hrows, arows, alpha, msg, stage, sem1, sem2):
        c = lax.axis_index("c")
        s = lax.axis_index("s")
        wid = s * NC + c
        row0 = s * NPT
        lane = lax.iota(i32, 16)
        hpat = (lane % 8) % H

        def zrow(i, carry):
            for kk in range(DW // 16):
                stage[i, pl.ds(kk * 16, 16)] = jnp.zeros((16,), f32)
            return carry
        lax.fori_loop(0, NPT, zrow, 0)

        @pl.when(c == 0)
        def _():
            pltpu.sync_copy(accinit.at[pl.ds(row0, NPT)], stage)

        pltpu.sync_copy(stage, acc_sh.at[pl.ds(row0, NPT)])
        plsc.subcore_barrier()

        def chunk(j, carry):
            base = wid * PERW + j * K
            pltpu.sync_copy(srci.at[pl.ds(base, K)], sidx)
            pltpu.sync_copy(dsti.at[pl.ds(base, K)], didx)
            cp1 = pltpu.async_copy(hext.at[sidx], hrows, sem1)
            cp2 = pltpu.async_copy(adstt.at[didx], arows, sem2)
            cp1.wait()
            cp2.wait()

            def alo(i, carry2):
                row = 2 * i + lane // 8
                a_s = plsc.load_gather(hrows, [row, 64 + hpat])
                a_d = plsc.load_gather(arows, [row, hpat])
                alpha[pl.ds(i * 16, 16)] = _leaky_exp(a_s + a_d)
                return carry2
            lax.fori_loop(0, K * 8 // 16, alo, 0)

            def mlo(e, carry2):
                for kk in range(4):
                    hv = hrows[e, pl.ds(kk * 16, 16)]
                    al = plsc.load_gather(alpha, [e * 8 + 2 * kk + lane // 8])
                    msg[e, pl.ds(kk * 16, 16)] = hv * al
                al4 = plsc.load_gather(alpha, [e * 8 + lane % 8])
                msg[e, pl.ds(64, 16)] = jnp.where(lane < 8, al4, 0.0)
                return carry2
            lax.fori_loop(0, K, mlo, 0)

            pltpu.sync_copy(msg, acc_sh.at[didx], add=True)
            return carry
        lax.fori_loop(0, CHUNKS, chunk, 0)

        plsc.subcore_barrier()
        pltpu.sync_copy(acc_sh.at[pl.ds(row0, NPT)], stage)
        pltpu.sync_copy(stage, out.at[pl.ds(c * NP + row0, NPT)])

    return kern


_sc_edge_l1 = _make_sc_edge(HEADS)
_sc_edge_l2 = _make_sc_edge(1)


def _tc_call(body, out_shapes):
    return pl.pallas_call(body, out_shape=out_shapes)


def kernel(x, edge_index, W1, att_src1, att_dst1, b1, W2, att_src2, att_dst2, b2):
    src = edge_index[0].astype(i32)
    dst = edge_index[1].astype(i32)

    eye8 = jnp.eye(8, dtype=f32)
    ams1 = (att_src1.reshape(8, 8)[:, :, None] * eye8[:, None, :]).reshape(64, 8)
    amd1 = (att_dst1.reshape(8, 8)[:, :, None] * eye8[:, None, :]).reshape(64, 8)
    ams2 = jnp.concatenate([att_src2.reshape(64, 1), jnp.zeros((64, 7), f32)], axis=1)
    amd2 = jnp.concatenate([att_dst2.reshape(64, 1), jnp.zeros((64, 7), f32)], axis=1)
    e8 = jnp.asarray(_E8)
    b64 = jnp.asarray(_B64)
    r8 = jnp.asarray(_R8)

    hext1, accinit1, adst1 = _tc_call(_tc1_body, (
        jax.ShapeDtypeStruct((NP, DW), f32),
        jax.ShapeDtypeStruct((NP, DW), f32),
        jax.ShapeDtypeStruct((NP, 16), f32),
    ))(x, W1, ams1, amd1, e8)

    acc1 = _sc_edge_l1(hext1, adst1, src, dst, accinit1)

    hext2, accinit2, adst2 = _tc_call(_tc2_body, (
        jax.ShapeDtypeStruct((NP, DW), f32),
        jax.ShapeDtypeStruct((NP, DW), f32),
        jax.ShapeDtypeStruct((NP, 16), f32),
    ))(acc1, b1.reshape(1, 64), W2, ams2, amd2, e8, b64, r8)

    acc2 = _sc_edge_l2(hext2, adst2, src, dst, accinit2)

    out = _tc_call(_tc3_body, jax.ShapeDtypeStruct((N, OUT_CH), f32))(
        acc2, b2.reshape(1, 64), b64)
    return out


# speedup vs baseline: 40.6685x; 1.0069x over previous
import functools

import numpy as np
import jax
import jax.numpy as jnp
from jax import lax
from jax.experimental import pallas as pl
from jax.experimental.pallas import tpu as pltpu
from jax.experimental.pallas import tpu_sc as plsc

N = 10000
NP = 10240
E = 320000
HEADS = 8
HID = 8
OUT_CH = 64
DW = 80
NC, NS = 2, 16
NW = NC * NS
K = 128
CHUNKS = 80
PERW = CHUNKS * K
EP = NW * PERW
NPT = NP // NS

f32 = jnp.float32
i32 = jnp.int32

_E8 = np.repeat(np.eye(8, dtype=np.float32), 8, axis=1)
_B64 = np.zeros((8, 64), np.float32)
_B64[0, :] = 1.0
_R8 = np.zeros((8, 8), np.float32)
_R8[0, :] = 1.0


def _leaky_exp(z):
    return jnp.exp(jnp.where(z >= 0, z, 0.2 * z))


def _padrows(a):
    return jnp.concatenate([a, jnp.zeros((NP - N, a.shape[1]), f32)], axis=0)


def _tc1_body(x_ref, w_ref, ams_ref, amd_ref, e8_ref,
              hext_ref, accinit_ref, adst_ref):
    h = jnp.dot(x_ref[...], w_ref[...], preferred_element_type=f32)
    a_s = jnp.dot(h, ams_ref[...], preferred_element_type=f32)
    a_d = jnp.dot(h, amd_ref[...], preferred_element_type=f32)
    w = _leaky_exp(a_s + a_d)
    w64 = jnp.dot(w, e8_ref[...], preferred_element_type=f32)
    z8 = jnp.zeros((N, 8), f32)
    hext_ref[...] = _padrows(jnp.concatenate([h, a_s, z8], axis=1))
    accinit_ref[...] = jnp.concatenate(
        [_padrows(jnp.concatenate([h * w64, w, z8], axis=1)),
         jnp.zeros((NP, DW), f32)], axis=0)
    adst_ref[...] = _padrows(jnp.concatenate([a_d, z8], axis=1))


def _tc2_body(acc_ref, b1_ref, w2_ref, ams_ref, amd_ref, e8_ref, b64_ref, r8_ref,
              hext_ref, accinit_ref, adst_ref):
    a = acc_ref[0:N, :] + acc_ref[NP:NP + N, :]
    den = jnp.dot(a[:, 64:72], e8_ref[...], preferred_element_type=f32)
    o1 = a[:, 0:64] / (den + 1e-16) + b1_ref[...]
    h1 = jnp.where(o1 > 0, o1, jnp.exp(o1) - 1.0)
    h2 = jnp.dot(h1, w2_ref[...], preferred_element_type=f32)
    a_s = jnp.dot(h2, ams_ref[...], preferred_element_type=f32)
    a_d = jnp.dot(h2, amd_ref[...], preferred_element_type=f32)
    w8 = _leaky_exp(a_s + a_d)
    wrep = jnp.dot(w8, r8_ref[...], preferred_element_type=f32)
    w64 = jnp.dot(w8, b64_ref[...], preferred_element_type=f32)
    z8 = jnp.zeros((N, 8), f32)
    hext_ref[...] = _padrows(jnp.concatenate([h2, a_s, z8], axis=1))
    accinit_ref[...] = jnp.concatenate(
        [_padrows(jnp.concatenate([h2 * w64, wrep, z8], axis=1)),
         jnp.zeros((NP, DW), f32)], axis=0)
    adst_ref[...] = _padrows(jnp.concatenate([a_d, z8], axis=1))


def _tc3_body(acc_ref, b2_ref, b64_ref, out_ref):
    a = acc_ref[0:N, :] + acc_ref[NP:NP + N, :]
    den = jnp.dot(a[:, 64:72], b64_ref[...], preferred_element_type=f32)
    o = a[:, 0:64] / (den + 1e-16) + b2_ref[...]
    m = jnp.max(o, axis=1, keepdims=True)
    ls = o - m
    out_ref[...] = ls - jnp.log(jnp.sum(jnp.exp(ls), axis=1, keepdims=True))


def _make_sc_edge(H):
    mesh = plsc.VectorSubcoreMesh(core_axis_name="c", subcore_axis_name="s")

    @functools.partial(
        pl.kernel,
        out_type=jax.ShapeDtypeStruct((NC * NP, DW), f32),
        mesh=mesh,
        compiler_params=pltpu.CompilerParams(
            needs_layout_passes=False,
            use_tc_tiling_on_sc=False,
        ),
        scratch_types=[
            pltpu.VMEM_SHARED((NP, DW), f32),
            pltpu.VMEM((CHUNKS, K), i32),
            pltpu.VMEM((CHUNKS, K), i32),
            pltpu.VMEM((K, DW), f32),
            pltpu.VMEM((K, DW), f32),
            pltpu.VMEM((K, 16), f32),
            pltpu.VMEM((K, 16), f32),
            pltpu.VMEM((K * 8,), f32),
            pltpu.VMEM((K, DW), f32),
            pltpu.VMEM((K, DW), f32),
            pltpu.SemaphoreType.DMA,
            pltpu.SemaphoreType.DMA,
            pltpu.SemaphoreType.DMA,
            pltpu.SemaphoreType.DMA,
        ],
    )
    def kern(hext, adstt, srci, dsti, accinit, out,
             acc_sh, sidx, didx, hr0, hr1, ar0, ar1, alpha, ms0, ms1,
             sg0, sg1, ss0, ss1):
        c = lax.axis_index("c")
        s = lax.axis_index("s")
        wid = s * NC + c
        row0 = s * NPT
        lane = lax.iota(i32, 16)
        sub = lax.shift_right_logical(lane, 3)
        lane8 = lax.bitwise_and(lane, 7)
        hpat = lane8 if H > 1 else jnp.zeros((16,), i32)

        hbuf = (hr0, hr1)
        abuf = (ar0, ar1)
        mbuf = (ms0, ms1)
        gsem = (sg0, sg1)
        ssem = (ss0, ss1)

        pltpu.sync_copy(accinit.at[pl.ds(c * NP + row0, NPT)],
                        acc_sh.at[pl.ds(row0, NPT)])
        pltpu.sync_copy(srci.at[wid], sidx)
        pltpu.sync_copy(dsti.at[wid], didx)
        plsc.subcore_barrier()

        def gathers(j, b):
            pltpu.async_copy(hext.at[sidx.at[j]], hbuf[b], gsem[b])
            pltpu.async_copy(adstt.at[didx.at[j]], abuf[b], gsem[b])

        gathers(0, 0)
        gathers(1, 1)

        def pair(j2, carry):
            for b in range(2):
                cur = 2 * j2 + b
                hb, ab, mb = hbuf[b], abuf[b], mbuf[b]
                pltpu.make_async_copy(hext.at[sidx.at[0]], hb, gsem[b]).wait()
                pltpu.make_async_copy(adstt.at[didx.at[0]], ab, gsem[b]).wait()
                @pl.when(j2 >= 1)
                def _():
                    pltpu.make_async_copy(
                        mb, acc_sh.at[didx.at[0]], ssem[b]).wait()

                def alo(i, carry2):
                    for u in range(4):
                        ii = 4 * i + u
                        row = 2 * ii + sub
                        a_s = plsc.load_gather(hb, [row, 64 + hpat])
                        a_d = plsc.load_gather(ab, [row, hpat])
                        alpha[pl.ds(ii * 16, 16)] = _leaky_exp(a_s + a_d)
                    return carry2
                lax.fori_loop(0, K * 8 // 16 // 4, alo, 0, unroll=True)

                def mlo(e4, carry2):
                    for u in range(4):
                        e = 4 * e4 + u
                        for kk in range(4):
                            hv = hb[e, pl.ds(kk * 16, 16)]
                            al = plsc.load_gather(
                                alpha, [e * 8 + 2 * kk + sub])
                            mb[e, pl.ds(kk * 16, 16)] = hv * al
                        al4 = plsc.load_gather(alpha, [e * 8 + lane8])
                        mb[e, pl.ds(64, 16)] = jnp.where(lane < 8, al4, 0.0)
                    return carry2
                lax.fori_loop(0, K // 4, mlo, 0, unroll=True)

                pltpu.async_copy(mb, acc_sh.at[didx.at[cur]], ssem[b],
                                 add=True)
                @pl.when(cur + 2 < CHUNKS)
                def _():
                    gathers(cur + 2, b)
            return carry
        lax.fori_loop(0, CHUNKS // 2, pair, 0)

        for b in range(2):
            pltpu.make_async_copy(mbuf[b], acc_sh.at[didx.at[0]],
                                  ssem[b]).wait()

        plsc.subcore_barrier()
        pltpu.sync_copy(acc_sh.at[pl.ds(row0, NPT)],
                        out.at[pl.ds(c * NP + row0, NPT)])

    return kern


_sc_edge_l1 = _make_sc_edge(HEADS)
_sc_edge_l2 = _make_sc_edge(1)


def _tc_call(body, out_shapes):
    return pl.pallas_call(body, out_shape=out_shapes)


def kernel(x, edge_index, W1, att_src1, att_dst1, b1, W2, att_src2, att_dst2, b2):
    src = edge_index[0].astype(i32)
    dst = edge_index[1].astype(i32)
    pad = jnp.full((EP - E,), NP - 1, i32)
    src3 = jnp.concatenate([src, pad]).reshape(NW, CHUNKS, K)
    dst3 = jnp.concatenate([dst, pad]).reshape(NW, CHUNKS, K)

    eye8 = jnp.eye(8, dtype=f32)
    ams1 = (att_src1.reshape(8, 8)[:, :, None] * eye8[:, None, :]).reshape(64, 8)
    amd1 = (att_dst1.reshape(8, 8)[:, :, None] * eye8[:, None, :]).reshape(64, 8)
    ams2 = jnp.concatenate([att_src2.reshape(64, 1), jnp.zeros((64, 7), f32)], axis=1)
    amd2 = jnp.concatenate([att_dst2.reshape(64, 1), jnp.zeros((64, 7), f32)], axis=1)
    e8 = jnp.asarray(_E8)
    b64 = jnp.asarray(_B64)
    r8 = jnp.asarray(_R8)

    hext1, accinit1, adst1 = _tc_call(_tc1_body, (
        jax.ShapeDtypeStruct((NP, DW), f32),
        jax.ShapeDtypeStruct((2 * NP, DW), f32),
        jax.ShapeDtypeStruct((NP, 16), f32),
    ))(x, W1, ams1, amd1, e8)

    acc1 = _sc_edge_l1(hext1, adst1, src3, dst3, accinit1)

    hext2, accinit2, adst2 = _tc_call(_tc2_body, (
        jax.ShapeDtypeStruct((NP, DW), f32),
        jax.ShapeDtypeStruct((2 * NP, DW), f32),
        jax.ShapeDtypeStruct((NP, 16), f32),
    ))(acc1, b1.reshape(1, 64), W2, ams2, amd2, e8, b64, r8)

    acc2 = _sc_edge_l2(hext2, adst2, src3, dst3, accinit2)

    out = _tc_call(_tc3_body, jax.ShapeDtypeStruct((N, OUT_CH), f32))(
        acc2, b2.reshape(1, 64), b64)
    return out


# speedup vs baseline: 50.0019x; 1.2295x over previous
import functools

import numpy as np
import jax
import jax.numpy as jnp
from jax import lax
from jax.experimental import pallas as pl
from jax.experimental.pallas import tpu as pltpu
from jax.experimental.pallas import tpu_sc as plsc

N = 10000
NP = 10240
E = 320000
HEADS = 8
HID = 8
OUT_CH = 64
DW = 80
NC, NS = 2, 16
NW = NC * NS
K = 128
CHUNKS = 80
PERW = CHUNKS * K
EP = NW * PERW
NPT = NP // NS

f32 = jnp.float32
i32 = jnp.int32

_E8 = np.repeat(np.eye(8, dtype=np.float32), 8, axis=1)
_B64 = np.zeros((8, 64), np.float32)
_B64[0, :] = 1.0
_R8 = np.zeros((8, 8), np.float32)
_R8[0, :] = 1.0


def _leaky_exp(z):
    return jnp.exp(jnp.where(z >= 0, z, 0.2 * z))


def _padrows(a):
    return jnp.concatenate([a, jnp.zeros((NP - N, a.shape[1]), f32)], axis=0)


def _tc1_body(x_ref, w_ref, ams_ref, amd_ref, e8_ref,
              hext_ref, accinit_ref, adst_ref):
    h = jnp.dot(x_ref[...], w_ref[...], preferred_element_type=f32)
    a_s = jnp.dot(h, ams_ref[...], preferred_element_type=f32)
    a_d = jnp.dot(h, amd_ref[...], preferred_element_type=f32)
    w = _leaky_exp(a_s + a_d)
    w64 = jnp.dot(w, e8_ref[...], preferred_element_type=f32)
    z8 = jnp.zeros((N, 8), f32)
    hext_ref[...] = _padrows(jnp.concatenate([h, a_s, z8], axis=1))
    accinit_ref[...] = jnp.concatenate(
        [_padrows(jnp.concatenate([h * w64, w, z8], axis=1)),
         jnp.zeros((NP, DW), f32)], axis=0)
    adst_ref[...] = _padrows(jnp.concatenate([a_d, z8], axis=1))


def _tc2_body(acc_ref, b1_ref, w2_ref, ams_ref, amd_ref, e8_ref, b64_ref, r8_ref,
              hext_ref, accinit_ref, adst_ref):
    a = acc_ref[0:N, :] + acc_ref[NP:NP + N, :]
    den = jnp.dot(a[:, 64:72], e8_ref[...], preferred_element_type=f32)
    o1 = a[:, 0:64] / (den + 1e-16) + b1_ref[...]
    h1 = jnp.where(o1 > 0, o1, jnp.exp(o1) - 1.0)
    h2 = jnp.dot(h1, w2_ref[...], preferred_element_type=f32)
    a_s = jnp.dot(h2, ams_ref[...], preferred_element_type=f32)
    a_d = jnp.dot(h2, amd_ref[...], preferred_element_type=f32)
    w8 = _leaky_exp(a_s + a_d)
    wrep = jnp.dot(w8, r8_ref[...], preferred_element_type=f32)
    w64 = jnp.dot(w8, b64_ref[...], preferred_element_type=f32)
    z8 = jnp.zeros((N, 8), f32)
    hext_ref[...] = _padrows(jnp.concatenate([h2, a_s, z8], axis=1))
    accinit_ref[...] = jnp.concatenate(
        [_padrows(jnp.concatenate([h2 * w64, wrep, z8], axis=1)),
         jnp.zeros((NP, DW), f32)], axis=0)
    adst_ref[...] = _padrows(jnp.concatenate([a_d, z8], axis=1))


def _tc3_body(acc_ref, b2_ref, b64_ref, out_ref):
    a = acc_ref[0:N, :] + acc_ref[NP:NP + N, :]
    den = jnp.dot(a[:, 64:72], b64_ref[...], preferred_element_type=f32)
    o = a[:, 0:64] / (den + 1e-16) + b2_ref[...]
    m = jnp.max(o, axis=1, keepdims=True)
    ls = o - m
    out_ref[...] = ls - jnp.log(jnp.sum(jnp.exp(ls), axis=1, keepdims=True))


def _make_sc_edge(H):
    mesh = plsc.VectorSubcoreMesh(core_axis_name="c", subcore_axis_name="s")

    @functools.partial(
        pl.kernel,
        out_type=jax.ShapeDtypeStruct((NC * NP, DW), f32),
        mesh=mesh,
        compiler_params=pltpu.CompilerParams(
            needs_layout_passes=False,
            use_tc_tiling_on_sc=False,
        ),
        scratch_types=[
            pltpu.VMEM_SHARED((NP, DW), f32),
            pltpu.VMEM((CHUNKS, K), i32),
            pltpu.VMEM((CHUNKS, K), i32),
            pltpu.VMEM((K, DW), f32),
            pltpu.VMEM((K, DW), f32),
            pltpu.VMEM((K, 16), f32),
            pltpu.VMEM((K, 16), f32),
            pltpu.VMEM((K * 8,), f32),
            pltpu.VMEM((K, DW), f32),
            pltpu.VMEM((K, DW), f32),
            pltpu.SemaphoreType.DMA,
            pltpu.SemaphoreType.DMA,
            pltpu.SemaphoreType.DMA,
            pltpu.SemaphoreType.DMA,
        ],
    )
    def kern(hext, adstt, srci, dsti, accinit, out,
             acc_sh, sidx, didx, hr0, hr1, ar0, ar1, alpha, ms0, ms1,
             sg0, sg1, ss0, ss1):
        c = lax.axis_index("c")
        s = lax.axis_index("s")
        wid = s * NC + c
        row0 = s * NPT
        lane = lax.iota(i32, 16)
        sub = lax.shift_right_logical(lane, 3)
        lane8 = lax.bitwise_and(lane, 7)
        hpat = lane8 if H > 1 else jnp.zeros((16,), i32)

        hbuf = (hr0, hr1)
        abuf = (ar0, ar1)
        mbuf = (ms0, ms1)
        gsem = (sg0, sg1)
        ssem = (ss0, ss1)

        pltpu.sync_copy(accinit.at[pl.ds(c * NP + row0, NPT)],
                        acc_sh.at[pl.ds(row0, NPT)])
        pltpu.sync_copy(srci.at[wid], sidx)
        pltpu.sync_copy(dsti.at[wid], didx)
        plsc.subcore_barrier()

        def gathers(j, b):
            pltpu.async_copy(hext.at[sidx.at[j]], hbuf[b], gsem[b])
            pltpu.async_copy(adstt.at[didx.at[j]], abuf[b], gsem[b])

        gathers(0, 0)
        gathers(1, 1)

        def pair(j2, carry):
            for b in range(2):
                cur = 2 * j2 + b
                hb, ab, mb = hbuf[b], abuf[b], mbuf[b]
                pltpu.make_async_copy(hext.at[sidx.at[0]], hb, gsem[b]).wait()
                pltpu.make_async_copy(adstt.at[didx.at[0]], ab, gsem[b]).wait()
                @pl.when(j2 >= 1)
                def _():
                    pltpu.make_async_copy(
                        mb, acc_sh.at[didx.at[0]], ssem[b]).wait()

                def alo(i, carry2):
                    for u in range(4):
                        ii = 4 * i + u
                        row = 2 * ii + sub
                        a_s = plsc.load_gather(hb, [row, 64 + hpat])
                        a_d = plsc.load_gather(ab, [row, hpat])
                        alpha[pl.ds(ii * 16, 16)] = _leaky_exp(a_s + a_d)
                    return carry2
                lax.fori_loop(0, K * 8 // 16 // 4, alo, 0, unroll=True)

                def mlo(e4, carry2):
                    for u in range(4):
                        e = 4 * e4 + u
                        for kk in range(4):
                            hv = hb[e, pl.ds(kk * 16, 16)]
                            al = plsc.load_gather(
                                alpha, [e * 8 + 2 * kk + sub])
                            mb[e, pl.ds(kk * 16, 16)] = hv * al
                        al4 = plsc.load_gather(alpha, [e * 8 + lane8])
                        mb[e, pl.ds(64, 16)] = jnp.where(lane < 8, al4, 0.0)
                    return carry2
                lax.fori_loop(0, K // 4, mlo, 0, unroll=True)

                pltpu.async_copy(mb, acc_sh.at[didx.at[cur]], ssem[b],
                                 add=True)
                @pl.when(cur + 2 < CHUNKS)
                def _():
                    gathers(cur + 2, b)
            return carry
        lax.fori_loop(0, CHUNKS // 2, pair, 0)

        for b in range(2):
            pltpu.make_async_copy(mbuf[b], acc_sh.at[didx.at[0]],
                                  ssem[b]).wait()

        plsc.subcore_barrier()
        pltpu.sync_copy(acc_sh.at[pl.ds(row0, NPT)],
                        out.at[pl.ds(c * NP + row0, NPT)])

    return kern


def _make_sc_edge_h1():
    mesh = plsc.VectorSubcoreMesh(core_axis_name="c", subcore_axis_name="s")

    @functools.partial(
        pl.kernel,
        out_type=jax.ShapeDtypeStruct((NC * NP, DW), f32),
        mesh=mesh,
        compiler_params=pltpu.CompilerParams(
            needs_layout_passes=False,
            use_tc_tiling_on_sc=False,
        ),
        scratch_types=[
            pltpu.VMEM_SHARED((NP, DW), f32),
            pltpu.VMEM((CHUNKS, K), i32),
            pltpu.VMEM((CHUNKS, K), i32),
            pltpu.VMEM((NP,), f32),
            pltpu.VMEM((K, DW), f32),
            pltpu.VMEM((K, DW), f32),
            pltpu.VMEM((K,), f32),
            pltpu.VMEM((K, DW), f32),
            pltpu.VMEM((K, DW), f32),
            pltpu.SemaphoreType.DMA,
            pltpu.SemaphoreType.DMA,
            pltpu.SemaphoreType.DMA,
            pltpu.SemaphoreType.DMA,
        ],
    )
    def kern(hext, adst1, srci, dsti, accinit, out,
             acc_sh, sidx, didx, adt, hr0, hr1, alpha, ms0, ms1,
             sg0, sg1, ss0, ss1):
        c = lax.axis_index("c")
        s = lax.axis_index("s")
        wid = s * NC + c
        row0 = s * NPT
        lane = lax.iota(i32, 16)
        c64 = lane * 0 + 64

        hbuf = (hr0, hr1)
        mbuf = (ms0, ms1)
        gsem = (sg0, sg1)
        ssem = (ss0, ss1)

        pltpu.sync_copy(accinit.at[pl.ds(c * NP + row0, NPT)],
                        acc_sh.at[pl.ds(row0, NPT)])
        pltpu.sync_copy(srci.at[wid], sidx)
        pltpu.sync_copy(dsti.at[wid], didx)
        pltpu.sync_copy(adst1, adt)
        plsc.subcore_barrier()

        def gathers(j, b):
            pltpu.async_copy(hext.at[sidx.at[j]], hbuf[b], gsem[b])

        gathers(0, 0)
        gathers(1, 1)

        def pair(j2, carry):
            for b in range(2):
                cur = 2 * j2 + b
                hb, mb = hbuf[b], mbuf[b]
                pltpu.make_async_copy(hext.at[sidx.at[0]], hb, gsem[b]).wait()

                @pl.when(j2 >= 1)
                def _():
                    pltpu.make_async_copy(
                        mb, acc_sh.at[didx.at[0]], ssem[b]).wait()

                def alo(i, carry2):
                    for u in range(2):
                        ii = 2 * i + u
                        dvec = didx[cur, pl.ds(ii * 16, 16)]
                        a_d = plsc.load_gather(adt, [dvec])
                        a_s = plsc.load_gather(hb, [ii * 16 + lane, c64])
                        alpha[pl.ds(ii * 16, 16)] = _leaky_exp(a_s + a_d)
                    return carry2
                lax.fori_loop(0, K // 16 // 2, alo, 0, unroll=True)

                def mlo(e4, carry2):
                    for u in range(4):
                        e = 4 * e4 + u
                        al = plsc.load_gather(alpha, [lane * 0 + e])
                        for kk in range(4):
                            hv = hb[e, pl.ds(kk * 16, 16)]
                            mb[e, pl.ds(kk * 16, 16)] = hv * al
                        mb[e, pl.ds(64, 16)] = jnp.where(lane < 8, al, 0.0)
                    return carry2
                lax.fori_loop(0, K // 4, mlo, 0, unroll=True)

                pltpu.async_copy(mb, acc_sh.at[didx.at[cur]], ssem[b],
                                 add=True)

                @pl.when(cur + 2 < CHUNKS)
                def _():
                    gathers(cur + 2, b)
            return carry
        lax.fori_loop(0, CHUNKS // 2, pair, 0)

        for b in range(2):
            pltpu.make_async_copy(mbuf[b], acc_sh.at[didx.at[0]],
                                  ssem[b]).wait()

        plsc.subcore_barrier()
        pltpu.sync_copy(acc_sh.at[pl.ds(row0, NPT)],
                        out.at[pl.ds(c * NP + row0, NPT)])

    return kern


_sc_edge_l1 = _make_sc_edge(HEADS)
_sc_edge_l2 = _make_sc_edge_h1()


def _tc_call(body, out_shapes):
    return pl.pallas_call(body, out_shape=out_shapes)


def kernel(x, edge_index, W1, att_src1, att_dst1, b1, W2, att_src2, att_dst2, b2):
    src = edge_index[0].astype(i32)
    dst = edge_index[1].astype(i32)
    pad = jnp.full((EP - E,), NP - 1, i32)
    src3 = jnp.concatenate([src, pad]).reshape(NW, CHUNKS, K)
    dst3 = jnp.concatenate([dst, pad]).reshape(NW, CHUNKS, K)

    eye8 = jnp.eye(8, dtype=f32)
    ams1 = (att_src1.reshape(8, 8)[:, :, None] * eye8[:, None, :]).reshape(64, 8)
    amd1 = (att_dst1.reshape(8, 8)[:, :, None] * eye8[:, None, :]).reshape(64, 8)
    ams2 = jnp.concatenate([att_src2.reshape(64, 1), jnp.zeros((64, 7), f32)], axis=1)
    amd2 = jnp.concatenate([att_dst2.reshape(64, 1), jnp.zeros((64, 7), f32)], axis=1)
    e8 = jnp.asarray(_E8)
    b64 = jnp.asarray(_B64)
    r8 = jnp.asarray(_R8)

    hext1, accinit1, adst1 = _tc_call(_tc1_body, (
        jax.ShapeDtypeStruct((NP, DW), f32),
        jax.ShapeDtypeStruct((2 * NP, DW), f32),
        jax.ShapeDtypeStruct((NP, 16), f32),
    ))(x, W1, ams1, amd1, e8)

    acc1 = _sc_edge_l1(hext1, adst1, src3, dst3, accinit1)

    hext2, accinit2, adst2 = _tc_call(_tc2_body, (
        jax.ShapeDtypeStruct((NP, DW), f32),
        jax.ShapeDtypeStruct((2 * NP, DW), f32),
        jax.ShapeDtypeStruct((NP, 16), f32),
    ))(acc1, b1.reshape(1, 64), W2, ams2, amd2, e8, b64, r8)

    acc2 = _sc_edge_l2(hext2, adst2[:, 0], src3, dst3, accinit2)

    out = _tc_call(_tc3_body, jax.ShapeDtypeStruct((N, OUT_CH), f32))(
        acc2, b2.reshape(1, 64), b64)
    return out


# speedup vs baseline: 52.8631x; 1.0572x over previous
import functools

import numpy as np
import jax
import jax.numpy as jnp
from jax import lax
from jax.experimental import pallas as pl
from jax.experimental.pallas import tpu as pltpu
from jax.experimental.pallas import tpu_sc as plsc

N = 10000
NP = 10240
E = 320000
HEADS = 8
HID = 8
OUT_CH = 64
DW = 80
NC, NS = 2, 16
NW = NC * NS
K = 64
CHUNKS = 160
PERW = CHUNKS * K
EP = NW * PERW
NPT = NP // NS

f32 = jnp.float32
i32 = jnp.int32

_E8 = np.repeat(np.eye(8, dtype=np.float32), 8, axis=1)
_B64 = np.zeros((8, 64), np.float32)
_B64[0, :] = 1.0
_R8 = np.zeros((8, 8), np.float32)
_R8[0, :] = 1.0


def _leaky_exp(z):
    return jnp.exp(jnp.where(z >= 0, z, 0.2 * z))


def _padrows(a):
    return jnp.concatenate([a, jnp.zeros((NP - N, a.shape[1]), f32)], axis=0)


def _tc1_body(x_ref, w_ref, ams_ref, amd_ref, e8_ref,
              hext_ref, accinit_ref, adst_ref):
    h = jnp.dot(x_ref[...], w_ref[...], preferred_element_type=f32)
    a_s = jnp.dot(h, ams_ref[...], preferred_element_type=f32)
    a_d = jnp.dot(h, amd_ref[...], preferred_element_type=f32)
    w = _leaky_exp(a_s + a_d)
    w64 = jnp.dot(w, e8_ref[...], preferred_element_type=f32)
    z8 = jnp.zeros((N, 8), f32)
    hext_ref[...] = _padrows(jnp.concatenate([h, a_s, z8], axis=1))
    accinit_ref[...] = jnp.concatenate(
        [_padrows(jnp.concatenate([h * w64, w, z8], axis=1)),
         jnp.zeros((NP, DW), f32)], axis=0)
    adst_ref[...] = _padrows(jnp.concatenate([a_d, z8], axis=1))


def _tc2_body(acc_ref, b1_ref, w2_ref, ams_ref, amd_ref, e8_ref, b64_ref, r8_ref,
              hext_ref, accinit_ref, adst_ref):
    a = acc_ref[0:N, :] + acc_ref[NP:NP + N, :]
    den = jnp.dot(a[:, 64:72], e8_ref[...], preferred_element_type=f32)
    o1 = a[:, 0:64] / (den + 1e-16) + b1_ref[...]
    h1 = jnp.where(o1 > 0, o1, jnp.exp(o1) - 1.0)
    h2 = jnp.dot(h1, w2_ref[...], preferred_element_type=f32)
    a_s = jnp.dot(h2, ams_ref[...], preferred_element_type=f32)
    a_d = jnp.dot(h2, amd_ref[...], preferred_element_type=f32)
    w8 = _leaky_exp(a_s + a_d)
    wrep = jnp.dot(w8, r8_ref[...], preferred_element_type=f32)
    w64 = jnp.dot(w8, b64_ref[...], preferred_element_type=f32)
    z8 = jnp.zeros((N, 8), f32)
    hext_ref[...] = _padrows(jnp.concatenate([h2, a_s, z8], axis=1))
    accinit_ref[...] = jnp.concatenate(
        [_padrows(jnp.concatenate([h2 * w64, wrep, z8], axis=1)),
         jnp.zeros((NP, DW), f32)], axis=0)
    adst_ref[...] = _padrows(jnp.concatenate([a_d, z8], axis=1))


def _tc3_body(acc_ref, b2_ref, b64_ref, out_ref):
    a = acc_ref[0:N, :] + acc_ref[NP:NP + N, :]
    den = jnp.dot(a[:, 64:72], b64_ref[...], preferred_element_type=f32)
    o = a[:, 0:64] / (den + 1e-16) + b2_ref[...]
    m = jnp.max(o, axis=1, keepdims=True)
    ls = o - m
    out_ref[...] = ls - jnp.log(jnp.sum(jnp.exp(ls), axis=1, keepdims=True))


def _make_sc_edge(H):
    mesh = plsc.VectorSubcoreMesh(core_axis_name="c", subcore_axis_name="s")

    @functools.partial(
        pl.kernel,
        out_type=jax.ShapeDtypeStruct((NC * NP, DW), f32),
        mesh=mesh,
        compiler_params=pltpu.CompilerParams(
            needs_layout_passes=False,
            use_tc_tiling_on_sc=False,
        ),
        scratch_types=[
            pltpu.VMEM_SHARED((NP, DW), f32),
            pltpu.VMEM((CHUNKS, K), i32),
            pltpu.VMEM((CHUNKS, K), i32),
            pltpu.VMEM((K, DW), f32),
            pltpu.VMEM((K, DW), f32),
            pltpu.VMEM((K, 16), f32),
            pltpu.VMEM((K, 16), f32),
            pltpu.VMEM((K * 8,), f32),
            pltpu.VMEM((K, DW), f32),
            pltpu.VMEM((K, DW), f32),
            pltpu.SemaphoreType.DMA,
            pltpu.SemaphoreType.DMA,
            pltpu.SemaphoreType.DMA,
            pltpu.SemaphoreType.DMA,
        ],
    )
    def kern(hext, adstt, srci, dsti, accinit, out,
             acc_sh, sidx, didx, hr0, hr1, ar0, ar1, alpha, ms0, ms1,
             sg0, sg1, ss0, ss1):
        c = lax.axis_index("c")
        s = lax.axis_index("s")
        wid = s * NC + c
        row0 = s * NPT
        lane = lax.iota(i32, 16)
        sub = lax.shift_right_logical(lane, 3)
        lane8 = lax.bitwise_and(lane, 7)
        hpat = lane8 if H > 1 else jnp.zeros((16,), i32)

        hbuf = (hr0, hr1)
        abuf = (ar0, ar1)
        mbuf = (ms0, ms1)
        gsem = (sg0, sg1)
        ssem = (ss0, ss1)

        pltpu.sync_copy(accinit.at[pl.ds(c * NP + row0, NPT)],
                        acc_sh.at[pl.ds(row0, NPT)])
        pltpu.sync_copy(srci.at[wid], sidx)
        pltpu.sync_copy(dsti.at[wid], didx)
        plsc.subcore_barrier()

        def gathers(j, b):
            pltpu.async_copy(hext.at[sidx.at[j]], hbuf[b], gsem[b])
            pltpu.async_copy(adstt.at[didx.at[j]], abuf[b], gsem[b])

        gathers(0, 0)
        gathers(1, 1)

        def pair(j2, carry):
            for b in range(2):
                cur = 2 * j2 + b
                hb, ab, mb = hbuf[b], abuf[b], mbuf[b]
                pltpu.make_async_copy(hext.at[sidx.at[0]], hb, gsem[b]).wait()
                pltpu.make_async_copy(adstt.at[didx.at[0]], ab, gsem[b]).wait()
                @pl.when(j2 >= 1)
                def _():
                    pltpu.make_async_copy(
                        mb, acc_sh.at[didx.at[0]], ssem[b]).wait()

                def alo(i, carry2):
                    for u in range(4):
                        ii = 4 * i + u
                        row = 2 * ii + sub
                        a_s = plsc.load_gather(hb, [row, 64 + hpat])
                        a_d = plsc.load_gather(ab, [row, hpat])
                        alpha[pl.ds(ii * 16, 16)] = _leaky_exp(a_s + a_d)
                    return carry2
                lax.fori_loop(0, K * 8 // 16 // 4, alo, 0, unroll=True)

                def mlo(e4, carry2):
                    for u in range(4):
                        e = 4 * e4 + u
                        for kk in range(4):
                            hv = hb[e, pl.ds(kk * 16, 16)]
                            al = plsc.load_gather(
                                alpha, [e * 8 + 2 * kk + sub])
                            mb[e, pl.ds(kk * 16, 16)] = hv * al
                        al4 = plsc.load_gather(alpha, [e * 8 + lane8])
                        mb[e, pl.ds(64, 16)] = jnp.where(lane < 8, al4, 0.0)
                    return carry2
                lax.fori_loop(0, K // 4, mlo, 0, unroll=True)

                pltpu.async_copy(mb, acc_sh.at[didx.at[cur]], ssem[b],
                                 add=True)
                @pl.when(cur + 2 < CHUNKS)
                def _():
                    gathers(cur + 2, b)
            return carry
        lax.fori_loop(0, CHUNKS // 2, pair, 0)

        for b in range(2):
            pltpu.make_async_copy(mbuf[b], acc_sh.at[didx.at[0]],
                                  ssem[b]).wait()

        plsc.subcore_barrier()
        pltpu.sync_copy(acc_sh.at[pl.ds(row0, NPT)],
                        out.at[pl.ds(c * NP + row0, NPT)])

    return kern


def _make_sc_edge_h1():
    mesh = plsc.VectorSubcoreMesh(core_axis_name="c", subcore_axis_name="s")

    @functools.partial(
        pl.kernel,
        out_type=jax.ShapeDtypeStruct((NC * NP, DW), f32),
        mesh=mesh,
        compiler_params=pltpu.CompilerParams(
            needs_layout_passes=False,
            use_tc_tiling_on_sc=False,
        ),
        scratch_types=[
            pltpu.VMEM_SHARED((NP, DW), f32),
            pltpu.VMEM((CHUNKS, K), i32),
            pltpu.VMEM((CHUNKS, K), i32),
            pltpu.VMEM((NP,), f32),
            pltpu.VMEM((K, DW), f32),
            pltpu.VMEM((K, DW), f32),
            pltpu.VMEM((K, DW), f32),
            pltpu.VMEM((K, DW), f32),
            pltpu.VMEM((K,), f32),
            pltpu.VMEM((K, DW), f32),
            pltpu.VMEM((K, DW), f32),
            pltpu.SemaphoreType.DMA,
            pltpu.SemaphoreType.DMA,
            pltpu.SemaphoreType.DMA,
            pltpu.SemaphoreType.DMA,
            pltpu.SemaphoreType.DMA,
            pltpu.SemaphoreType.DMA,
        ],
    )
    def kern(hext, adst1, srci, dsti, accinit, out,
             acc_sh, sidx, didx, adt, hr0, hr1, hr2, hr3, alpha, ms0, ms1,
             sg0, sg1, sg2, sg3, ss0, ss1):
        c = lax.axis_index("c")
        s = lax.axis_index("s")
        wid = s * NC + c
        row0 = s * NPT
        lane = lax.iota(i32, 16)
        c64 = lane * 0 + 64

        hbuf = (hr0, hr1, hr2, hr3)
        mbuf = (ms0, ms1)
        gsem = (sg0, sg1, sg2, sg3)
        ssem = (ss0, ss1)

        pltpu.sync_copy(accinit.at[pl.ds(c * NP + row0, NPT)],
                        acc_sh.at[pl.ds(row0, NPT)])
        pltpu.sync_copy(srci.at[wid], sidx)
        pltpu.sync_copy(dsti.at[wid], didx)
        pltpu.sync_copy(adst1, adt)
        plsc.subcore_barrier()

        def gathers(j, b):
            pltpu.async_copy(hext.at[sidx.at[j]], hbuf[b], gsem[b])

        for b in range(4):
            gathers(b, b)

        def pair(j4, carry):
            for b in range(4):
                cur = 4 * j4 + b
                hb, mb = hbuf[b], mbuf[b % 2]
                pltpu.make_async_copy(hext.at[sidx.at[0]], hb, gsem[b]).wait()

                @pl.when(cur >= 2)
                def _():
                    pltpu.make_async_copy(
                        mb, acc_sh.at[didx.at[0]], ssem[b % 2]).wait()

                def alo(i, carry2):
                    for u in range(2):
                        ii = 2 * i + u
                        dvec = didx[cur, pl.ds(ii * 16, 16)]
                        a_d = plsc.load_gather(adt, [dvec])
                        a_s = plsc.load_gather(hb, [ii * 16 + lane, c64])
                        alpha[pl.ds(ii * 16, 16)] = _leaky_exp(a_s + a_d)
                    return carry2
                lax.fori_loop(0, K // 16 // 2, alo, 0, unroll=True)

                def mlo(e4, carry2):
                    for u in range(4):
                        e = 4 * e4 + u
                        al = plsc.load_gather(alpha, [lane * 0 + e])
                        for kk in range(4):
                            hv = hb[e, pl.ds(kk * 16, 16)]
                            mb[e, pl.ds(kk * 16, 16)] = hv * al
                        mb[e, pl.ds(64, 16)] = jnp.where(lane < 8, al, 0.0)
                    return carry2
                lax.fori_loop(0, K // 4, mlo, 0, unroll=True)

                pltpu.async_copy(mb, acc_sh.at[didx.at[cur]], ssem[b % 2],
                                 add=True)

                @pl.when(cur + 4 < CHUNKS)
                def _():
                    gathers(cur + 4, b)
            return carry
        lax.fori_loop(0, CHUNKS // 4, pair, 0)

        for b in range(2):
            pltpu.make_async_copy(mbuf[b], acc_sh.at[didx.at[0]],
                                  ssem[b]).wait()

        plsc.subcore_barrier()
        pltpu.sync_copy(acc_sh.at[pl.ds(row0, NPT)],
                        out.at[pl.ds(c * NP + row0, NPT)])

    return kern


_sc_edge_l1 = _make_sc_edge(HEADS)
_sc_edge_l2 = _make_sc_edge_h1()


def _tc_call(body, out_shapes):
    return pl.pallas_call(body, out_shape=out_shapes)


def kernel(x, edge_index, W1, att_src1, att_dst1, b1, W2, att_src2, att_dst2, b2):
    src = edge_index[0].astype(i32)
    dst = edge_index[1].astype(i32)
    pad = jnp.full((EP - E,), NP - 1, i32)
    src3 = jnp.concatenate([src, pad]).reshape(NW, CHUNKS, K)
    dst3 = jnp.concatenate([dst, pad]).reshape(NW, CHUNKS, K)

    eye8 = jnp.eye(8, dtype=f32)
    ams1 = (att_src1.reshape(8, 8)[:, :, None] * eye8[:, None, :]).reshape(64, 8)
    amd1 = (att_dst1.reshape(8, 8)[:, :, None] * eye8[:, None, :]).reshape(64, 8)
    ams2 = jnp.concatenate([att_src2.reshape(64, 1), jnp.zeros((64, 7), f32)], axis=1)
    amd2 = jnp.concatenate([att_dst2.reshape(64, 1), jnp.zeros((64, 7), f32)], axis=1)
    e8 = jnp.asarray(_E8)
    b64 = jnp.asarray(_B64)
    r8 = jnp.asarray(_R8)

    hext1, accinit1, adst1 = _tc_call(_tc1_body, (
        jax.ShapeDtypeStruct((NP, DW), f32),
        jax.ShapeDtypeStruct((2 * NP, DW), f32),
        jax.ShapeDtypeStruct((NP, 16), f32),
    ))(x, W1, ams1, amd1, e8)

    acc1 = _sc_edge_l1(hext1, adst1, src3, dst3, accinit1)

    hext2, accinit2, adst2 = _tc_call(_tc2_body, (
        jax.ShapeDtypeStruct((NP, DW), f32),
        jax.ShapeDtypeStruct((2 * NP, DW), f32),
        jax.ShapeDtypeStruct((NP, 16), f32),
    ))(acc1, b1.reshape(1, 64), W2, ams2, amd2, e8, b64, r8)

    acc2 = _sc_edge_l2(hext2, adst2[:, 0], src3, dst3, accinit2)

    out = _tc_call(_tc3_body, jax.ShapeDtypeStruct((N, OUT_CH), f32))(
        acc2, b2.reshape(1, 64), b64)
    return out


# speedup vs baseline: 52.9878x; 1.0024x over previous
import functools

import numpy as np
import jax
import jax.numpy as jnp
from jax import lax
from jax.experimental import pallas as pl
from jax.experimental.pallas import tpu as pltpu
from jax.experimental.pallas import tpu_sc as plsc

N = 10000
NP = 10240
E = 320000
HEADS = 8
HID = 8
OUT_CH = 64
DW = 80
NC, NS = 2, 16
NW = NC * NS
K = 64
CHUNKS = 160
PERW = CHUNKS * K
EP = NW * PERW
NPT = NP // NS

f32 = jnp.float32
i32 = jnp.int32

_E8 = np.repeat(np.eye(8, dtype=np.float32), 8, axis=1)
_B64 = np.zeros((8, 64), np.float32)
_B64[0, :] = 1.0
_R8 = np.zeros((8, 8), np.float32)
_R8[0, :] = 1.0


def _leaky_exp(z):
    return jnp.exp(jnp.where(z >= 0, z, 0.2 * z))


def _padrows(a):
    return jnp.concatenate([a, jnp.zeros((NP - N, a.shape[1]), f32)], axis=0)


def _tc1_body(x_ref, w_ref, ams_ref, amd_ref, e8_ref,
              hext_ref, accinit_ref, adst_ref):
    h = jnp.dot(x_ref[...], w_ref[...], preferred_element_type=f32)
    a_s = jnp.dot(h, ams_ref[...], preferred_element_type=f32)
    a_d = jnp.dot(h, amd_ref[...], preferred_element_type=f32)
    w = _leaky_exp(a_s + a_d)
    w64 = jnp.dot(w, e8_ref[...], preferred_element_type=f32)
    z8 = jnp.zeros((N, 8), f32)
    hext_ref[...] = _padrows(jnp.concatenate([h, a_s, z8], axis=1))
    accinit_ref[...] = jnp.concatenate(
        [_padrows(jnp.concatenate([h * w64, w, z8], axis=1)),
         jnp.zeros((NP, DW), f32)], axis=0)
    adst_ref[...] = _padrows(jnp.concatenate([a_d, z8], axis=1))


def _tc2_body(acc_ref, b1_ref, w2_ref, ams_ref, amd_ref, e8_ref, b64_ref, r8_ref,
              hext_ref, accinit_ref, adst_ref):
    a = acc_ref[0:N, :] + acc_ref[NP:NP + N, :]
    den = jnp.dot(a[:, 64:72], e8_ref[...], preferred_element_type=f32)
    o1 = a[:, 0:64] / (den + 1e-16) + b1_ref[...]
    h1 = jnp.where(o1 > 0, o1, jnp.exp(o1) - 1.0)
    h2 = jnp.dot(h1, w2_ref[...], preferred_element_type=f32)
    a_s = jnp.dot(h2, ams_ref[...], preferred_element_type=f32)
    a_d = jnp.dot(h2, amd_ref[...], preferred_element_type=f32)
    w8 = _leaky_exp(a_s + a_d)
    wrep = jnp.dot(w8, r8_ref[...], preferred_element_type=f32)
    w64 = jnp.dot(w8, b64_ref[...], preferred_element_type=f32)
    z8 = jnp.zeros((N, 8), f32)
    hext_ref[...] = _padrows(jnp.concatenate([h2, a_s, z8], axis=1))
    accinit_ref[...] = jnp.concatenate(
        [_padrows(jnp.concatenate([h2 * w64, wrep, z8], axis=1)),
         jnp.zeros((NP, DW), f32)], axis=0)
    adst_ref[...] = _padrows(jnp.concatenate([a_d, z8], axis=1))


def _tc3_body(acc_ref, b2_ref, b64_ref, out_ref):
    a = acc_ref[0:N, :] + acc_ref[NP:NP + N, :]
    den = jnp.dot(a[:, 64:72], b64_ref[...], preferred_element_type=f32)
    o = a[:, 0:64] / (den + 1e-16) + b2_ref[...]
    m = jnp.max(o, axis=1, keepdims=True)
    ls = o - m
    out_ref[...] = ls - jnp.log(jnp.sum(jnp.exp(ls), axis=1, keepdims=True))


def _make_sc_edge(H):
    mesh = plsc.VectorSubcoreMesh(core_axis_name="c", subcore_axis_name="s")

    @functools.partial(
        pl.kernel,
        out_type=jax.ShapeDtypeStruct((NC * NP, DW), f32),
        mesh=mesh,
        compiler_params=pltpu.CompilerParams(
            needs_layout_passes=False,
            use_tc_tiling_on_sc=False,
        ),
        scratch_types=[
            pltpu.VMEM_SHARED((NP, DW), f32),
            pltpu.VMEM((CHUNKS, K), i32),
            pltpu.VMEM((CHUNKS, K), i32),
            pltpu.VMEM((K, DW), f32),
            pltpu.VMEM((K, DW), f32),
            pltpu.VMEM((K, DW), f32),
            pltpu.VMEM((K, DW), f32),
            pltpu.VMEM((K, 16), f32),
            pltpu.VMEM((K, 16), f32),
            pltpu.VMEM((K, 16), f32),
            pltpu.VMEM((K, 16), f32),
            pltpu.VMEM((K * 8,), f32),
            pltpu.VMEM((K, DW), f32),
            pltpu.VMEM((K, DW), f32),
            pltpu.SemaphoreType.DMA,
            pltpu.SemaphoreType.DMA,
            pltpu.SemaphoreType.DMA,
            pltpu.SemaphoreType.DMA,
            pltpu.SemaphoreType.DMA,
            pltpu.SemaphoreType.DMA,
        ],
    )
    def kern(hext, adstt, srci, dsti, accinit, out,
             acc_sh, sidx, didx, hr0, hr1, hr2, hr3, ar0, ar1, ar2, ar3,
             alpha, ms0, ms1, sg0, sg1, sg2, sg3, ss0, ss1):
        c = lax.axis_index("c")
        s = lax.axis_index("s")
        wid = s * NC + c
        row0 = s * NPT
        lane = lax.iota(i32, 16)
        sub = lax.shift_right_logical(lane, 3)
        lane8 = lax.bitwise_and(lane, 7)
        hpat = lane8 if H > 1 else jnp.zeros((16,), i32)

        hbuf = (hr0, hr1, hr2, hr3)
        abuf = (ar0, ar1, ar2, ar3)
        mbuf = (ms0, ms1)
        gsem = (sg0, sg1, sg2, sg3)
        ssem = (ss0, ss1)

        pltpu.sync_copy(accinit.at[pl.ds(c * NP + row0, NPT)],
                        acc_sh.at[pl.ds(row0, NPT)])
        pltpu.sync_copy(srci.at[wid], sidx)
        pltpu.sync_copy(dsti.at[wid], didx)
        plsc.subcore_barrier()

        def gathers(j, b):
            pltpu.async_copy(hext.at[sidx.at[j]], hbuf[b], gsem[b])
            pltpu.async_copy(adstt.at[didx.at[j]], abuf[b], gsem[b])

        for b in range(4):
            gathers(b, b)

        def pair(j4, carry):
            for b in range(4):
                cur = 4 * j4 + b
                hb, ab, mb = hbuf[b], abuf[b], mbuf[b % 2]
                pltpu.make_async_copy(hext.at[sidx.at[0]], hb, gsem[b]).wait()
                pltpu.make_async_copy(adstt.at[didx.at[0]], ab, gsem[b]).wait()
                @pl.when(cur >= 2)
                def _():
                    pltpu.make_async_copy(
                        mb, acc_sh.at[didx.at[0]], ssem[b % 2]).wait()

                def alo(i, carry2):
                    for u in range(4):
                        ii = 4 * i + u
                        row = 2 * ii + sub
                        a_s = plsc.load_gather(hb, [row, 64 + hpat])
                        a_d = plsc.load_gather(ab, [row, hpat])
                        alpha[pl.ds(ii * 16, 16)] = _leaky_exp(a_s + a_d)
                    return carry2
                lax.fori_loop(0, K * 8 // 16 // 4, alo, 0, unroll=True)

                def mlo(e4, carry2):
                    for u in range(4):
                        e = 4 * e4 + u
                        for kk in range(4):
                            hv = hb[e, pl.ds(kk * 16, 16)]
                            al = plsc.load_gather(
                                alpha, [e * 8 + 2 * kk + sub])
                            mb[e, pl.ds(kk * 16, 16)] = hv * al
                        al4 = plsc.load_gather(alpha, [e * 8 + lane8])
                        mb[e, pl.ds(64, 16)] = jnp.where(lane < 8, al4, 0.0)
                    return carry2
                lax.fori_loop(0, K // 4, mlo, 0, unroll=True)

                pltpu.async_copy(mb, acc_sh.at[didx.at[cur]], ssem[b % 2],
                                 add=True)
                @pl.when(cur + 4 < CHUNKS)
                def _():
                    gathers(cur + 4, b)
            return carry
        lax.fori_loop(0, CHUNKS // 4, pair, 0)

        for b in range(2):
            pltpu.make_async_copy(mbuf[b], acc_sh.at[didx.at[0]],
                                  ssem[b]).wait()

        plsc.subcore_barrier()
        pltpu.sync_copy(acc_sh.at[pl.ds(row0, NPT)],
                        out.at[pl.ds(c * NP + row0, NPT)])

    return kern


def _make_sc_edge_h1():
    mesh = plsc.VectorSubcoreMesh(core_axis_name="c", subcore_axis_name="s")

    @functools.partial(
        pl.kernel,
        out_type=jax.ShapeDtypeStruct((NC * NP, DW), f32),
        mesh=mesh,
        compiler_params=pltpu.CompilerParams(
            needs_layout_passes=False,
            use_tc_tiling_on_sc=False,
        ),
        scratch_types=[
            pltpu.VMEM_SHARED((NP, DW), f32),
            pltpu.VMEM((CHUNKS, K), i32),
            pltpu.VMEM((CHUNKS, K), i32),
            pltpu.VMEM((NP,), f32),
            pltpu.VMEM((K, DW), f32),
            pltpu.VMEM((K, DW), f32),
            pltpu.VMEM((K, DW), f32),
            pltpu.VMEM((K, DW), f32),
            pltpu.VMEM((K,), f32),
            pltpu.VMEM((K, DW), f32),
            pltpu.VMEM((K, DW), f32),
            pltpu.SemaphoreType.DMA,
            pltpu.SemaphoreType.DMA,
            pltpu.SemaphoreType.DMA,
            pltpu.SemaphoreType.DMA,
            pltpu.SemaphoreType.DMA,
            pltpu.SemaphoreType.DMA,
        ],
    )
    def kern(hext, adst1, srci, dsti, accinit, out,
             acc_sh, sidx, didx, adt, hr0, hr1, hr2, hr3, alpha, ms0, ms1,
             sg0, sg1, sg2, sg3, ss0, ss1):
        c = lax.axis_index("c")
        s = lax.axis_index("s")
        wid = s * NC + c
        row0 = s * NPT
        lane = lax.iota(i32, 16)
        c64 = lane * 0 + 64

        hbuf = (hr0, hr1, hr2, hr3)
        mbuf = (ms0, ms1)
        gsem = (sg0, sg1, sg2, sg3)
        ssem = (ss0, ss1)

        pltpu.sync_copy(accinit.at[pl.ds(c * NP + row0, NPT)],
                        acc_sh.at[pl.ds(row0, NPT)])
        pltpu.sync_copy(srci.at[wid], sidx)
        pltpu.sync_copy(dsti.at[wid], didx)
        pltpu.sync_copy(adst1, adt)
        plsc.subcore_barrier()

        def gathers(j, b):
            pltpu.async_copy(hext.at[sidx.at[j]], hbuf[b], gsem[b])

        for b in range(4):
            gathers(b, b)

        def pair(j4, carry):
            for b in range(4):
                cur = 4 * j4 + b
                hb, mb = hbuf[b], mbuf[b % 2]
                pltpu.make_async_copy(hext.at[sidx.at[0]], hb, gsem[b]).wait()

                @pl.when(cur >= 2)
                def _():
                    pltpu.make_async_copy(
                        mb, acc_sh.at[didx.at[0]], ssem[b % 2]).wait()

                def alo(i, carry2):
                    for u in range(2):
                        ii = 2 * i + u
                        dvec = didx[cur, pl.ds(ii * 16, 16)]
                        a_d = plsc.load_gather(adt, [dvec])
                        a_s = plsc.load_gather(hb, [ii * 16 + lane, c64])
                        alpha[pl.ds(ii * 16, 16)] = _leaky_exp(a_s + a_d)
                    return carry2
                lax.fori_loop(0, K // 16 // 2, alo, 0, unroll=True)

                def mlo(e4, carry2):
                    for u in range(4):
                        e = 4 * e4 + u
                        al = plsc.load_gather(alpha, [lane * 0 + e])
                        for kk in range(4):
                            hv = hb[e, pl.ds(kk * 16, 16)]
                            mb[e, pl.ds(kk * 16, 16)] = hv * al
                        mb[e, pl.ds(64, 16)] = jnp.where(lane < 8, al, 0.0)
                    return carry2
                lax.fori_loop(0, K // 4, mlo, 0, unroll=True)

                pltpu.async_copy(mb, acc_sh.at[didx.at[cur]], ssem[b % 2],
                                 add=True)

                @pl.when(cur + 4 < CHUNKS)
                def _():
                    gathers(cur + 4, b)
            return carry
        lax.fori_loop(0, CHUNKS // 4, pair, 0)

        for b in range(2):
            pltpu.make_async_copy(mbuf[b], acc_sh.at[didx.at[0]],
                                  ssem[b]).wait()

        plsc.subcore_barrier()
        pltpu.sync_copy(acc_sh.at[pl.ds(row0, NPT)],
                        out.at[pl.ds(c * NP + row0, NPT)])

    return kern


_sc_edge_l1 = _make_sc_edge(HEADS)
_sc_edge_l2 = _make_sc_edge_h1()


def _tc_call(body, out_shapes):
    return pl.pallas_call(body, out_shape=out_shapes)


def kernel(x, edge_index, W1, att_src1, att_dst1, b1, W2, att_src2, att_dst2, b2):
    src = edge_index[0].astype(i32)
    dst = edge_index[1].astype(i32)
    pad = jnp.full((EP - E,), NP - 1, i32)
    src3 = jnp.concatenate([src, pad]).reshape(NW, CHUNKS, K)
    dst3 = jnp.concatenate([dst, pad]).reshape(NW, CHUNKS, K)

    eye8 = jnp.eye(8, dtype=f32)
    ams1 = (att_src1.reshape(8, 8)[:, :, None] * eye8[:, None, :]).reshape(64, 8)
    amd1 = (att_dst1.reshape(8, 8)[:, :, None] * eye8[:, None, :]).reshape(64, 8)
    ams2 = jnp.concatenate([att_src2.reshape(64, 1), jnp.zeros((64, 7), f32)], axis=1)
    amd2 = jnp.concatenate([att_dst2.reshape(64, 1), jnp.zeros((64, 7), f32)], axis=1)
    e8 = jnp.asarray(_E8)
    b64 = jnp.asarray(_B64)
    r8 = jnp.asarray(_R8)

    hext1, accinit1, adst1 = _tc_call(_tc1_body, (
        jax.ShapeDtypeStruct((NP, DW), f32),
        jax.ShapeDtypeStruct((2 * NP, DW), f32),
        jax.ShapeDtypeStruct((NP, 16), f32),
    ))(x, W1, ams1, amd1, e8)

    acc1 = _sc_edge_l1(hext1, adst1, src3, dst3, accinit1)

    hext2, accinit2, adst2 = _tc_call(_tc2_body, (
        jax.ShapeDtypeStruct((NP, DW), f32),
        jax.ShapeDtypeStruct((2 * NP, DW), f32),
        jax.ShapeDtypeStruct((NP, 16), f32),
    ))(acc1, b1.reshape(1, 64), W2, ams2, amd2, e8, b64, r8)

    acc2 = _sc_edge_l2(hext2, adst2[:, 0], src3, dst3, accinit2)

    out = _tc_call(_tc3_body, jax.ShapeDtypeStruct((N, OUT_CH), f32))(
        acc2, b2.reshape(1, 64), b64)
    return out
